# baseline (speedup 1.0000x reference)
import functools

import jax
import jax.numpy as jnp
from jax import lax
from jax.experimental import pallas as pl
from jax.experimental.pallas import tpu as pltpu
from jax.experimental.pallas import tpu_sc as plsc

_NC = 2
_NS = 16
_NW = _NC * _NS

_SUB = 128
_GRP = 1024


def _sc_gather(table, x_lin, l0, lcount, b):
    n = lcount * b
    v, d = table.shape
    n_sub = _GRP // _SUB
    n_grp = b // _GRP
    k_max = (lcount + _NW - 1) // _NW
    rpl = b // 8

    mesh = plsc.VectorSubcoreMesh(core_axis_name="c", subcore_axis_name="s")

    @functools.partial(
        pl.kernel,
        mesh=mesh,
        compiler_params=pltpu.CompilerParams(
            use_tc_tiling_on_sc=False, needs_layout_passes=False),
        out_type=jax.ShapeDtypeStruct((n, d), jnp.float32),
        scratch_types=[
            pltpu.VMEM((b,), jnp.int32),
            pltpu.VMEM((b,), jnp.int32),
            pltpu.VMEM((2, _GRP, d), jnp.float32),
            pltpu.SemaphoreType.DMA,
            pltpu.SemaphoreType.DMA,
        ],
    )
    def gather_kernel(table_hbm, x_hbm, out_hbm, xv, pidx, rows_v, sem, wsem):
        wid = lax.axis_index("s") * _NC + lax.axis_index("c")
        iota16 = lax.iota(jnp.int32, 16)
        pending = []

        for k in range(k_max):
            lk = jnp.minimum(wid + _NW * k, lcount - 1)
            base = lk * b
            pltpu.sync_copy(x_hbm.at[pl.ds((l0 + lk) * b, b)], xv)

            def perm_body(m, carry):
                bi = 16 * m
                vv = xv[pl.ds(bi, 16)]
                vt = ((vv & ~1023) | ((vv & 127) << 3)
                      | ((vv >> 7) & 7))
                pos = (8 * (bi % rpl) + bi // rpl) + 8 * iota16
                plsc.store_scatter(pidx, [pos], vt)
                return carry

            lax.fori_loop(0, b // 16, perm_body, 0, unroll=8)

            for g in range(n_grp):
                buf = (k * n_grp + g) % 2
                if len(pending) >= 2:
                    pending.pop(0).wait()
                descs = [
                    pltpu.async_copy(
                        table_hbm.at[
                            pidx.at[pl.ds((g * n_sub + j) * _SUB, _SUB)]],
                        rows_v.at[buf, pl.ds(j * _SUB, _SUB)],
                        sem,
                    )
                    for j in range(n_sub)
                ]
                for desc in descs:
                    desc.wait()
                pending.append(pltpu.async_copy(
                    rows_v.at[buf],
                    out_hbm.at[pl.ds(base + g * _GRP, _GRP)], wsem))

        for desc in pending:
            desc.wait()

    return gather_kernel(table, x_lin)


def _tc_mlp_t(emb, W1, b1, W2, b2, l, b):
    n, d = emb.shape
    h = W1.shape[1]
    o = W2.shape[1]
    assert n == l * b

    blk_l = 8
    g = 128 // d
    rpl = b // g
    assert l % blk_l == 0 and b % g == 0

    emb128 = emb.reshape(n * d // 128, 128)

    w1p = (jnp.eye(g, dtype=jnp.float32)[:, None, :, None]
           * jnp.transpose(W1)[None, :, None, :]).reshape(g * h, g * d)
    w2p = (jnp.eye(g, dtype=jnp.float32)[None, :, :, None]
           * jnp.transpose(W2)[:, None, None, :]).reshape(g * o, g * h)
    b1p = jnp.tile(b1, g).reshape(g * h, 1)
    b2p = jnp.repeat(b2, g).reshape(g * o, 1)

    def body(emb_ref, w1_ref, b1_ref, w2_ref, b2_ref, out_ref):
        e = emb_ref[...]
        hid = lax.dot_general(
            w1_ref[...], e, (((1,), (1,)), ((), ())),
            preferred_element_type=jnp.float32)
        hid = jnp.maximum(hid + b1_ref[...], 0.0)
        logits = jnp.dot(
            w2_ref[...], hid, preferred_element_type=jnp.float32)
        logits = logits + b2_ref[...]
        lg = jnp.reshape(logits, (o, g, blk_l * rpl))
        m = jnp.max(lg, axis=0, keepdims=True)
        s = lg - m
        lse = jnp.log(jnp.sum(jnp.exp(s), axis=0, keepdims=True))
        r3 = s - lse
        for j in range(blk_l):
            for q in range(g):
                out_ref[:, j, pl.ds(q * rpl, rpl)] = (
                    r3[:, q, j * rpl:(j + 1) * rpl])

    return pl.pallas_call(
        body,
        grid=(l // blk_l,),
        in_specs=[
            pl.BlockSpec((blk_l * rpl, 128), lambda i: (i, 0)),
            pl.BlockSpec((g * h, g * d), lambda i: (0, 0)),
            pl.BlockSpec((g * h, 1), lambda i: (0, 0)),
            pl.BlockSpec((g * o, g * h), lambda i: (0, 0)),
            pl.BlockSpec((g * o, 1), lambda i: (0, 0)),
        ],
        out_specs=pl.BlockSpec((o, blk_l, b), lambda i: (0, i, 0)),
        out_shape=jax.ShapeDtypeStruct((o, l, b), jnp.float32),
    )(emb128, w1p, b1p, w2p, b2p)


def _sc_warm():
    mesh = plsc.VectorSubcoreMesh(core_axis_name="c", subcore_axis_name="s")

    @functools.partial(
        pl.kernel,
        mesh=mesh,
        compiler_params=pltpu.CompilerParams(
            use_tc_tiling_on_sc=False, needs_layout_passes=False),
        out_type=jax.ShapeDtypeStruct((_NW, 16), jnp.float32),
        scratch_types=[pltpu.VMEM((16,), jnp.float32)],
    )
    def warm_kernel(out_hbm, tmp):
        wid = lax.axis_index("s") * _NC + lax.axis_index("c")
        tmp[...] = jnp.zeros((16,), jnp.float32)
        pltpu.sync_copy(tmp, out_hbm.at[wid])

    return warm_kernel()


def _tc_table_pack(tableT, kp, grid):
    d, v = tableT.shape
    g = 128 // d
    rows = 128 * grid * kp

    def body(in_ref, out_ref):
        for p in range(kp):
            cat = jnp.concatenate(
                [in_ref[:, pl.ds(1024 * p + 128 * j, 128)] for j in range(g)],
                axis=0)
            out_ref[pl.ds(128 * p, 128), :] = jnp.transpose(cat)

    return pl.pallas_call(
        body,
        grid=(grid,),
        in_specs=[pl.BlockSpec((d, 1024 * kp), lambda i: (0, i))],
        out_specs=pl.BlockSpec((128 * kp, g * d), lambda i: (i, 0)),
        out_shape=jax.ShapeDtypeStruct((rows, g * d), jnp.float32),
    )(tableT)


def kernel(x, table, W1, b1, W2, b2):
    b, l = x.shape
    d = table.shape[1]
    g = 128 // d
    kp, grid = 14, 70
    packed = _tc_table_pack(jnp.transpose(table), kp, grid)
    table_lin = packed.reshape(packed.shape[0] * g, d)
    x_lin = jnp.transpose(x).reshape(-1).astype(jnp.int32)
    chunks = [(0, 32), (32, 80), (112, 56), (168, 32)]
    warm = _sc_warm()
    b1w = b1 + warm[0, 0]
    embs = [_sc_gather(table_lin, x_lin, l0, lc, b) for l0, lc in chunks]
    outs = [_tc_mlp_t(e, W1, b1w, W2, b2, lc, b)
            for e, (l0, lc) in zip(embs, chunks)]
    out3 = jnp.concatenate(outs, axis=1)
    return jnp.transpose(out3, (2, 1, 0))


# speedup vs baseline: 1.0074x; 1.0074x over previous
import functools

import jax
import jax.numpy as jnp
from jax import lax
from jax.experimental import pallas as pl
from jax.experimental.pallas import tpu as pltpu
from jax.experimental.pallas import tpu_sc as plsc

_NC = 2
_NS = 16
_NW = _NC * _NS

_SUB = 128
_GRP = 1024


def _sc_gather(table, x_lin, l0, lcount, b):
    n = lcount * b
    v, d = table.shape
    n_sub = _GRP // _SUB
    n_grp = b // _GRP
    k_max = (lcount + _NW - 1) // _NW
    rpl = b // 8

    mesh = plsc.VectorSubcoreMesh(core_axis_name="c", subcore_axis_name="s")

    @functools.partial(
        pl.kernel,
        mesh=mesh,
        compiler_params=pltpu.CompilerParams(
            use_tc_tiling_on_sc=False, needs_layout_passes=False),
        out_type=jax.ShapeDtypeStruct((n, d), jnp.float32),
        scratch_types=[
            pltpu.VMEM((b,), jnp.int32),
            pltpu.VMEM((b,), jnp.int32),
            pltpu.VMEM((2, _GRP, d), jnp.float32),
            pltpu.SemaphoreType.DMA,
            pltpu.SemaphoreType.DMA,
        ],
    )
    def gather_kernel(table_hbm, x_hbm, out_hbm, xv, pidx, rows_v, sem, wsem):
        wid = lax.axis_index("s") * _NC + lax.axis_index("c")
        iota16 = lax.iota(jnp.int32, 16)
        pending = []

        for k in range(k_max):
            lk = jnp.minimum(wid + _NW * k, lcount - 1)
            base = lk * b
            pltpu.sync_copy(x_hbm.at[pl.ds((l0 + lk) * b, b)], xv)

            def perm_body(m, carry):
                bi = 16 * m
                vv = xv[pl.ds(bi, 16)]
                vt = ((vv & ~1023) | ((vv & 127) << 3)
                      | ((vv >> 7) & 7))
                pos = (8 * (bi % rpl) + bi // rpl) + 8 * iota16
                plsc.store_scatter(pidx, [pos], vt)
                return carry

            lax.fori_loop(0, b // 16, perm_body, 0, unroll=8)

            for g in range(n_grp):
                buf = (k * n_grp + g) % 2
                if len(pending) >= 2:
                    pending.pop(0).wait()
                descs = [
                    pltpu.async_copy(
                        table_hbm.at[
                            pidx.at[pl.ds((g * n_sub + j) * _SUB, _SUB)]],
                        rows_v.at[buf, pl.ds(j * _SUB, _SUB)],
                        sem,
                    )
                    for j in range(n_sub)
                ]
                for desc in descs:
                    desc.wait()
                pending.append(pltpu.async_copy(
                    rows_v.at[buf],
                    out_hbm.at[pl.ds(base + g * _GRP, _GRP)], wsem))

        for desc in pending:
            desc.wait()

    return gather_kernel(table, x_lin)


def _tc_mlp_t(emb, W1, b1, W2, b2, l, b):
    n, d = emb.shape
    h = W1.shape[1]
    o = W2.shape[1]
    assert n == l * b

    blk_l = 8
    g = 128 // d
    rpl = b // g
    assert l % blk_l == 0 and b % g == 0

    emb128 = emb.reshape(n * d // 128, 128)

    w1p = (jnp.eye(g, dtype=jnp.float32)[:, None, :, None]
           * jnp.transpose(W1)[None, :, None, :]).reshape(g * h, g * d)
    w2p = (jnp.eye(g, dtype=jnp.float32)[None, :, :, None]
           * jnp.transpose(W2)[:, None, None, :]).reshape(g * o, g * h)
    b1p = jnp.tile(b1, g).reshape(g * h, 1)
    b2p = jnp.repeat(b2, g).reshape(g * o, 1)

    def body(emb_ref, w1_ref, b1_ref, w2_ref, b2_ref, out_ref):
        e = emb_ref[...]
        hid = lax.dot_general(
            w1_ref[...], e, (((1,), (1,)), ((), ())),
            preferred_element_type=jnp.float32)
        hid = jnp.maximum(hid + b1_ref[...], 0.0)
        logits = jnp.dot(
            w2_ref[...], hid, preferred_element_type=jnp.float32)
        logits = logits + b2_ref[...]
        lg = jnp.reshape(logits, (o, g, blk_l * rpl))
        m = jnp.max(lg, axis=0, keepdims=True)
        s = lg - m
        lse = jnp.log(jnp.sum(jnp.exp(s), axis=0, keepdims=True))
        r3 = s - lse
        for j in range(blk_l):
            for q in range(g):
                out_ref[:, j, pl.ds(q * rpl, rpl)] = (
                    r3[:, q, j * rpl:(j + 1) * rpl])

    return pl.pallas_call(
        body,
        grid=(l // blk_l,),
        in_specs=[
            pl.BlockSpec((blk_l * rpl, 128), lambda i: (i, 0)),
            pl.BlockSpec((g * h, g * d), lambda i: (0, 0)),
            pl.BlockSpec((g * h, 1), lambda i: (0, 0)),
            pl.BlockSpec((g * o, g * h), lambda i: (0, 0)),
            pl.BlockSpec((g * o, 1), lambda i: (0, 0)),
        ],
        out_specs=pl.BlockSpec((o, blk_l, b), lambda i: (0, i, 0)),
        out_shape=jax.ShapeDtypeStruct((o, l, b), jnp.float32),
    )(emb128, w1p, b1p, w2p, b2p)


def _tc_table_pack(tableT, kp, grid):
    d, v = tableT.shape
    g = 128 // d
    rows = 128 * grid * kp

    def body(in_ref, out_ref):
        for p in range(kp):
            cat = jnp.concatenate(
                [in_ref[:, pl.ds(1024 * p + 128 * j, 128)] for j in range(g)],
                axis=0)
            out_ref[pl.ds(128 * p, 128), :] = jnp.transpose(cat)

    return pl.pallas_call(
        body,
        grid=(grid,),
        in_specs=[pl.BlockSpec((d, 1024 * kp), lambda i: (0, i))],
        out_specs=pl.BlockSpec((128 * kp, g * d), lambda i: (i, 0)),
        out_shape=jax.ShapeDtypeStruct((rows, g * d), jnp.float32),
    )(tableT)


def kernel(x, table, W1, b1, W2, b2):
    b, l = x.shape
    d = table.shape[1]
    g = 128 // d
    kp, grid = 14, 70
    packed = _tc_table_pack(jnp.transpose(table), kp, grid)
    table_lin = packed.reshape(packed.shape[0] * g, d)
    x_lin = jnp.transpose(x).reshape(-1).astype(jnp.int32)
    chunks = [(0, 32), (32, 80), (112, 56), (168, 32)]
    embs = [_sc_gather(table_lin, x_lin, l0, lc, b) for l0, lc in chunks]
    outs = [_tc_mlp_t(e, W1, b1, W2, b2, lc, b)
            for e, (l0, lc) in zip(embs, chunks)]
    out3 = jnp.concatenate(outs, axis=1)
    return jnp.transpose(out3, (2, 1, 0))
